# baseline (speedup 1.0000x reference)
import functools

import jax
import jax.numpy as jnp
from jax import lax
from jax.experimental import pallas as pl
from jax.experimental.pallas import tpu as pltpu
from jax.experimental.pallas import tpu_sc as plsc

_EOS_ID = 2
_NEG_INF = 1.0e7

_BATCH = 32
_BEAM = 4
_VOCAB = 100000
_K2 = 8

_CH = 2000
_NCH = _VOCAB // _CH
_U = 5
_NEG = -3.0e38
_BIGI = 2**31 - 1


def _splat_f(x):
    return jnp.full((16,), x, jnp.float32)


def _splat_i(x):
    return jnp.full((16,), x, jnp.int32)


_NLANES = 5


def _worker_id():
    return lax.axis_index("s") * 2 + lax.axis_index("c")


def _put(buf, idx, x, iota16, dtype=jnp.float32):
    plsc.store_scatter(buf, [jnp.full((16,), idx, jnp.int32)],
                       jnp.full((16,), x, dtype), mask=iota16 == 0)


_SEC = 20000
_NSEC = _VOCAB // _SEC
_CH_PER_SEC = _SEC // _CH


def _sc_body(logits, out_v, out_f, rowbuf, cmaxv, cmaxs, stage_v, stage_f,
             sem0, sem1):
    b = _worker_id()
    iota16 = lax.iota(jnp.int32, 16)
    nvec = _CH // 16

    for beam in range(_BEAM):
        rbase = (b * _BEAM + beam) * _VOCAB
        pltpu.async_copy(logits.at[pl.ds(rbase, _SEC)],
                         rowbuf.at[pl.ds(0, _SEC)], sem0)

        def chunk_body(c, g16):
            base = c * _CH
            accs = [_splat_f(_NEG) for _ in range(_NLANES)]
            for i in range(nvec):
                accs[i % _NLANES] = jnp.maximum(
                    accs[i % _NLANES], rowbuf[pl.ds(base + i * 16, 16)])
            m16 = accs[0]
            for a in accs[1:]:
                m16 = jnp.maximum(m16, a)
            cmaxv[pl.ds(c * 16, 16)] = m16
            return jnp.maximum(g16, m16)

        def sec_body(s, g16):
            sbase = s * _SEC

            @pl.when(s % 2 == 0)
            def _():
                pltpu.make_async_copy(
                    logits.at[pl.ds(rbase + sbase, _SEC)],
                    rowbuf.at[pl.ds(sbase, _SEC)], sem0).wait()

            @pl.when(s % 2 == 1)
            def _():
                pltpu.make_async_copy(
                    logits.at[pl.ds(rbase + sbase, _SEC)],
                    rowbuf.at[pl.ds(sbase, _SEC)], sem1).wait()

            nbase = sbase + _SEC

            @pl.when(jnp.logical_and(s + 1 < _NSEC, (s + 1) % 2 == 0))
            def _():
                pltpu.async_copy(logits.at[pl.ds(rbase + nbase, _SEC)],
                                 rowbuf.at[pl.ds(nbase, _SEC)], sem0)

            @pl.when(jnp.logical_and(s + 1 < _NSEC, (s + 1) % 2 == 1))
            def _():
                pltpu.async_copy(logits.at[pl.ds(rbase + nbase, _SEC)],
                                 rowbuf.at[pl.ds(nbase, _SEC)], sem1)

            return lax.fori_loop(s * _CH_PER_SEC, (s + 1) * _CH_PER_SEC,
                                 chunk_body, g16)

        lax.fori_loop(0, _NSEC, sec_body, _splat_f(_NEG))

        for g in range(4):
            acc = _splat_f(_NEG)
            for l in range(16):
                gi = (iota16 + g * 16) * 16 + l
                acc = jnp.maximum(acc, plsc.load_gather(cmaxv, [gi]))
            if g == 3:
                acc = jnp.where(iota16 + g * 16 < _NCH, acc, _NEG)
            cmaxs[pl.ds(g * 16, 16)] = acc

        for k in range(_K2):
            m16 = _splat_f(_NEG)
            am16 = _splat_i(0)
            for t in range(4):
                v = cmaxs[pl.ds(t * 16, 16)]
                idx16 = iota16 + t * 16
                better = v > m16
                m16 = jnp.where(better, v, m16)
                am16 = jnp.where(better, idx16, am16)
            bm = jnp.max(m16)
            bc = jnp.min(jnp.where(m16 == bm, am16, _BIGI))
            base = bc * _CH
            bm16 = _splat_f(bm)

            def scan_pos(t, best16):
                off = base + t * (16 * _U)
                for u in range(_U):
                    v = rowbuf[pl.ds(off + u * 16, 16)]
                    pos16 = iota16 + (off + u * 16)
                    best16 = jnp.minimum(
                        best16, jnp.where(v == bm16, pos16, _BIGI))
                return best16

            best16 = lax.fori_loop(0, nvec // _U, scan_pos, _splat_i(_BIGI))
            pos = jnp.min(best16)

            lane = pos % 16
            vecbase = pos - lane
            vv = rowbuf[pl.ds(vecbase, 16)]
            rowbuf[pl.ds(vecbase, 16)] = jnp.where(iota16 == lane, _NEG, vv)

            def rescan(t, m):
                off = base + t * (16 * _U)
                for u in range(_U):
                    m = jnp.maximum(m, rowbuf[pl.ds(off + u * 16, 16)])
                return m

            nm16 = lax.fori_loop(0, nvec // _U, rescan, _splat_f(_NEG))
            _put(cmaxs, bc, jnp.max(nm16), iota16)

            _put(stage_v, beam * _K2 + k, bm, iota16)
            _put(stage_f, beam * _K2 + k, beam * _VOCAB + pos, iota16,
                 jnp.int32)

    pltpu.sync_copy(stage_v, out_v.at[b])
    pltpu.sync_copy(stage_f, out_f.at[b])


def _sc_topk(logits):
    mesh = plsc.VectorSubcoreMesh(core_axis_name="c", subcore_axis_name="s",
                                  num_cores=2, num_subcores=16)
    fn = pl.kernel(
        _sc_body,
        out_type=[
            jax.ShapeDtypeStruct((_BATCH, 2 * _K2 * 2), jnp.float32),
            jax.ShapeDtypeStruct((_BATCH, 2 * _K2 * 2), jnp.int32),
        ],
        mesh=mesh,
        scratch_types=[
            pltpu.VMEM((_VOCAB,), jnp.float32),
            pltpu.VMEM((1024,), jnp.float32),
            pltpu.VMEM((64,), jnp.float32),
            pltpu.VMEM((2 * _K2 * 2,), jnp.float32),
            pltpu.VMEM((2 * _K2 * 2,), jnp.int32),
            pltpu.SemaphoreType.DMA,
            pltpu.SemaphoreType.DMA,
        ],
        compiler_params=pltpu.CompilerParams(needs_layout_passes=False),
    )
    return fn(logits)


def _lse_kernel(x_ref, lse_ref):
    x = x_ref[0]
    beam, V = x.shape
    m = jnp.max(x, axis=1, keepdims=True)
    s = jnp.sum(jnp.exp(x - m), axis=1, keepdims=True)
    lse_ref[0] = (m + jnp.log(s)).reshape(1, beam)


def _lse(logits):
    batch, beam, V = logits.shape
    out = pl.pallas_call(
        _lse_kernel,
        grid=(batch,),
        in_specs=[pl.BlockSpec((1, beam, V), lambda b: (b, 0, 0))],
        out_specs=pl.BlockSpec((1, 1, beam), lambda b: (b, 0, 0)),
        out_shape=jax.ShapeDtypeStruct((batch, 1, beam), jnp.float32),
    )(logits)
    return out.reshape(batch, beam)


def _tail_kernel(v_ref, f_ref, lse_ref, alp_ref, seq_ref, seq_out_ref, lp_out_ref):
    v = v_ref[...]
    f = f_ref[...]
    lse = lse_ref[...]
    alp = alp_ref[...]
    seq = seq_ref[...]

    n32 = 2 * _K2 * 2
    col = lax.broadcasted_iota(jnp.int32, (_BATCH, n32), 1)
    bcol = col // _K2
    Lx = jnp.zeros((_BATCH, n32), jnp.float32)
    Ax = jnp.zeros((_BATCH, n32), jnp.float32)
    for k in range(_BEAM):
        m = bcol == k
        Lx = jnp.where(m, lse[:, k:k + 1], Lx)
        Ax = jnp.where(m, alp[:, k:k + 1], Ax)
    score = v - Lx + Ax

    NEGF = jnp.float32(-3.0e38)
    work = score
    cand_val = jnp.zeros((_BATCH, _K2), jnp.float32)
    cand_flat = jnp.zeros((_BATCH, _K2), jnp.int32)
    col8 = lax.broadcasted_iota(jnp.int32, (_BATCH, _K2), 1)
    for j in range(_K2):
        m = jnp.max(work, axis=1, keepdims=True)
        selflat = jnp.min(jnp.where(work == m, f, _BIGI), axis=1, keepdims=True)
        work = jnp.where(f == selflat, NEGF, work)
        cand_val = jnp.where(col8 == j, m, cand_val)
        cand_flat = jnp.where(col8 == j, selflat, cand_flat)

    topk_id = cand_flat % _VOCAB
    topk_beam = cand_flat // _VOCAB

    ts = jnp.zeros((_BATCH, _K2, seq.shape[2]), jnp.int32)
    bsel = topk_beam[:, :, None]
    for k in range(_BEAM):
        ts = jnp.where(jnp.broadcast_to(bsel == k, ts.shape),
                       jnp.broadcast_to(seq[:, k:k + 1, :], ts.shape), ts)
    topk_seq = jnp.concatenate([ts, topk_id[:, :, None]], axis=2)

    fin = (topk_id == _EOS_ID).astype(jnp.float32)
    masked = cand_val + fin * jnp.float32(-_NEG_INF)

    work2 = masked
    out_lp = jnp.zeros((_BATCH, _BEAM), jnp.float32)
    col4 = lax.broadcasted_iota(jnp.int32, (_BATCH, _BEAM), 1)
    out_seq = jnp.zeros((_BATCH, _BEAM, topk_seq.shape[2]), jnp.int32)
    row4 = lax.broadcasted_iota(jnp.int32, (_BATCH, _BEAM, 1), 1)
    for k in range(_BEAM):
        m2 = jnp.max(work2, axis=1, keepdims=True)
        selpos = jnp.min(jnp.where(work2 == m2, col8, _BIGI), axis=1,
                         keepdims=True)
        work2 = jnp.where(col8 == selpos, NEGF, work2)
        out_lp = jnp.where(col4 == k, m2, out_lp)
        rowk = jnp.zeros((_BATCH, 1, topk_seq.shape[2]), jnp.int32)
        for j in range(_K2):
            rowk = jnp.where(
                jnp.broadcast_to(selpos[:, :, None] == j, rowk.shape),
                topk_seq[:, j:j + 1, :], rowk)
        out_seq = jnp.where(jnp.broadcast_to(row4 == k, out_seq.shape),
                            jnp.broadcast_to(rowk, out_seq.shape), out_seq)

    seq_out_ref[...] = out_seq
    lp_out_ref[...] = out_lp


def kernel(logits, alive_log_probs, alive_seq):
    batch, beam, V = logits.shape
    cur_len = alive_seq.shape[2]

    v, f = _sc_topk(logits.reshape(-1))
    lse = _lse(logits)

    seq_out, lp_out = pl.pallas_call(
        _tail_kernel,
        out_shape=[
            jax.ShapeDtypeStruct((batch, beam, cur_len + 1), jnp.int32),
            jax.ShapeDtypeStruct((batch, beam), jnp.float32),
        ],
    )(v, f, lse, alive_log_probs, alive_seq)
    return seq_out, lp_out


# speedup vs baseline: 1.6794x; 1.6794x over previous
import functools

import jax
import jax.numpy as jnp
from jax import lax
from jax.experimental import pallas as pl
from jax.experimental.pallas import tpu as pltpu
from jax.experimental.pallas import tpu_sc as plsc

_EOS_ID = 2
_NEG_INF = 1.0e7

_BATCH = 32
_BEAM = 4
_VOCAB = 100000
_K2 = 8

_CH = 2000
_NCH = _VOCAB // _CH
_U = 5
_NEG = -3.0e38
_BIGI = 2**31 - 1


def _splat_f(x):
    return jnp.full((16,), x, jnp.float32)


def _splat_i(x):
    return jnp.full((16,), x, jnp.int32)


_NLANES = 5


def _worker_id():
    return lax.axis_index("s") * 2 + lax.axis_index("c")


def _put(buf, idx, x, iota16, dtype=jnp.float32):
    plsc.store_scatter(buf, [jnp.full((16,), idx, jnp.int32)],
                       jnp.full((16,), x, dtype), mask=iota16 == 0)


_SEC = 20000
_NSEC = _VOCAB // _SEC
_CH_PER_SEC = _SEC // _CH


def _sc_body(logits, out_v, out_f, rowbuf, cmaxv, cmaxs, stage_v, stage_f):
    b = _worker_id()
    iota16 = lax.iota(jnp.int32, 16)
    nvec = _CH // 16

    for beam in range(_BEAM):
        pltpu.sync_copy(logits.at[b, beam], rowbuf)

        def chunk_body(c, g16):
            base = c * _CH
            accs = [_splat_f(_NEG) for _ in range(_NLANES)]
            for i in range(nvec):
                accs[i % _NLANES] = jnp.maximum(
                    accs[i % _NLANES], rowbuf[pl.ds(base + i * 16, 16)])
            m16 = accs[0]
            for a in accs[1:]:
                m16 = jnp.maximum(m16, a)
            cmaxv[pl.ds(c * 16, 16)] = m16
            return jnp.maximum(g16, m16)

        lax.fori_loop(0, _NCH, chunk_body, _splat_f(_NEG))

        for g in range(4):
            acc = _splat_f(_NEG)
            for l in range(16):
                gi = (iota16 + g * 16) * 16 + l
                acc = jnp.maximum(acc, plsc.load_gather(cmaxv, [gi]))
            if g == 3:
                acc = jnp.where(iota16 + g * 16 < _NCH, acc, _NEG)
            cmaxs[pl.ds(g * 16, 16)] = acc

        for k in range(_K2):
            m16 = _splat_f(_NEG)
            am16 = _splat_i(0)
            for t in range(4):
                v = cmaxs[pl.ds(t * 16, 16)]
                idx16 = iota16 + t * 16
                better = v > m16
                m16 = jnp.where(better, v, m16)
                am16 = jnp.where(better, idx16, am16)
            bm = jnp.max(m16)
            bc = jnp.min(jnp.where(m16 == bm, am16, _BIGI))
            base = bc * _CH
            bm16 = _splat_f(bm)

            def scan_pos(t, best16):
                off = base + t * (16 * _U)
                for u in range(_U):
                    v = rowbuf[pl.ds(off + u * 16, 16)]
                    pos16 = iota16 + (off + u * 16)
                    best16 = jnp.minimum(
                        best16, jnp.where(v == bm16, pos16, _BIGI))
                return best16

            best16 = lax.fori_loop(0, nvec // _U, scan_pos, _splat_i(_BIGI))
            pos = jnp.min(best16)

            lane = pos % 16
            vecbase = pos - lane
            vv = rowbuf[pl.ds(vecbase, 16)]
            rowbuf[pl.ds(vecbase, 16)] = jnp.where(iota16 == lane, _NEG, vv)

            def rescan(t, m):
                off = base + t * (16 * _U)
                for u in range(_U):
                    m = jnp.maximum(m, rowbuf[pl.ds(off + u * 16, 16)])
                return m

            nm16 = lax.fori_loop(0, nvec // _U, rescan, _splat_f(_NEG))
            _put(cmaxs, bc, jnp.max(nm16), iota16)

            _put(stage_v, beam * _K2 + k, bm, iota16)
            _put(stage_f, beam * _K2 + k, beam * _VOCAB + pos, iota16,
                 jnp.int32)

    pltpu.sync_copy(stage_v, out_v.at[b])
    pltpu.sync_copy(stage_f, out_f.at[b])


def _sc_topk(logits):
    mesh = plsc.VectorSubcoreMesh(core_axis_name="c", subcore_axis_name="s",
                                  num_cores=2, num_subcores=16)
    fn = pl.kernel(
        _sc_body,
        out_type=[
            jax.ShapeDtypeStruct((_BATCH, 2 * _K2 * 2), jnp.float32),
            jax.ShapeDtypeStruct((_BATCH, 2 * _K2 * 2), jnp.int32),
        ],
        mesh=mesh,
        scratch_types=[
            pltpu.VMEM((_VOCAB,), jnp.float32),
            pltpu.VMEM((1024,), jnp.float32),
            pltpu.VMEM((64,), jnp.float32),
            pltpu.VMEM((2 * _K2 * 2,), jnp.float32),
            pltpu.VMEM((2 * _K2 * 2,), jnp.int32),
        ],
        compiler_params=pltpu.CompilerParams(needs_layout_passes=False),
    )
    return fn(logits)


def _lse_kernel(x_ref, lse_ref):
    x = x_ref[0]
    beam, V = x.shape
    m = jnp.max(x, axis=1, keepdims=True)
    s = jnp.sum(jnp.exp(x - m), axis=1, keepdims=True)
    lse_ref[0] = (m + jnp.log(s)).reshape(1, beam)


def _lse(logits):
    batch, beam, V = logits.shape
    out = pl.pallas_call(
        _lse_kernel,
        grid=(batch,),
        in_specs=[pl.BlockSpec((1, beam, V), lambda b: (b, 0, 0))],
        out_specs=pl.BlockSpec((1, 1, beam), lambda b: (b, 0, 0)),
        out_shape=jax.ShapeDtypeStruct((batch, 1, beam), jnp.float32),
    )(logits)
    return out.reshape(batch, beam)


def _tail_kernel(v_ref, f_ref, lse_ref, alp_ref, seq_ref, seq_out_ref, lp_out_ref):
    v = v_ref[...]
    f = f_ref[...]
    lse = lse_ref[...]
    alp = alp_ref[...]
    seq = seq_ref[...]

    n32 = 2 * _K2 * 2
    col = lax.broadcasted_iota(jnp.int32, (_BATCH, n32), 1)
    bcol = col // _K2
    Lx = jnp.zeros((_BATCH, n32), jnp.float32)
    Ax = jnp.zeros((_BATCH, n32), jnp.float32)
    for k in range(_BEAM):
        m = bcol == k
        Lx = jnp.where(m, lse[:, k:k + 1], Lx)
        Ax = jnp.where(m, alp[:, k:k + 1], Ax)
    score = v - Lx + Ax

    NEGF = jnp.float32(-3.0e38)
    work = score
    cand_val = jnp.zeros((_BATCH, _K2), jnp.float32)
    cand_flat = jnp.zeros((_BATCH, _K2), jnp.int32)
    col8 = lax.broadcasted_iota(jnp.int32, (_BATCH, _K2), 1)
    for j in range(_K2):
        m = jnp.max(work, axis=1, keepdims=True)
        selflat = jnp.min(jnp.where(work == m, f, _BIGI), axis=1, keepdims=True)
        work = jnp.where(f == selflat, NEGF, work)
        cand_val = jnp.where(col8 == j, m, cand_val)
        cand_flat = jnp.where(col8 == j, selflat, cand_flat)

    topk_id = cand_flat % _VOCAB
    topk_beam = cand_flat // _VOCAB

    ts = jnp.zeros((_BATCH, _K2, seq.shape[2]), jnp.int32)
    bsel = topk_beam[:, :, None]
    for k in range(_BEAM):
        ts = jnp.where(jnp.broadcast_to(bsel == k, ts.shape),
                       jnp.broadcast_to(seq[:, k:k + 1, :], ts.shape), ts)
    topk_seq = jnp.concatenate([ts, topk_id[:, :, None]], axis=2)

    fin = (topk_id == _EOS_ID).astype(jnp.float32)
    masked = cand_val + fin * jnp.float32(-_NEG_INF)

    work2 = masked
    out_lp = jnp.zeros((_BATCH, _BEAM), jnp.float32)
    col4 = lax.broadcasted_iota(jnp.int32, (_BATCH, _BEAM), 1)
    out_seq = jnp.zeros((_BATCH, _BEAM, topk_seq.shape[2]), jnp.int32)
    row4 = lax.broadcasted_iota(jnp.int32, (_BATCH, _BEAM, 1), 1)
    for k in range(_BEAM):
        m2 = jnp.max(work2, axis=1, keepdims=True)
        selpos = jnp.min(jnp.where(work2 == m2, col8, _BIGI), axis=1,
                         keepdims=True)
        work2 = jnp.where(col8 == selpos, NEGF, work2)
        out_lp = jnp.where(col4 == k, m2, out_lp)
        rowk = jnp.zeros((_BATCH, 1, topk_seq.shape[2]), jnp.int32)
        for j in range(_K2):
            rowk = jnp.where(
                jnp.broadcast_to(selpos[:, :, None] == j, rowk.shape),
                topk_seq[:, j:j + 1, :], rowk)
        out_seq = jnp.where(jnp.broadcast_to(row4 == k, out_seq.shape),
                            jnp.broadcast_to(rowk, out_seq.shape), out_seq)

    seq_out_ref[...] = out_seq
    lp_out_ref[...] = out_lp


def kernel(logits, alive_log_probs, alive_seq):
    batch, beam, V = logits.shape
    cur_len = alive_seq.shape[2]

    v, f = _sc_topk(logits)
    lse = _lse(logits)

    seq_out, lp_out = pl.pallas_call(
        _tail_kernel,
        out_shape=[
            jax.ShapeDtypeStruct((batch, beam, cur_len + 1), jnp.int32),
            jax.ShapeDtypeStruct((batch, beam), jnp.float32),
        ],
    )(v, f, lse, alive_log_probs, alive_seq)
    return seq_out, lp_out
